# Initial kernel scaffold; baseline (speedup 1.0000x reference)
import jax
import jax.numpy as jnp
from jax.experimental import pallas as pl


def kernel(uids, iids, pos, neg, adj_rows, adj_cols, adj_vals, E_u_0, E_i_0, u_mul_s, v_mul_s, ut, vt):
    raise NotImplementedError("write your pallas kernel here")


# speedup vs baseline: 1.1120x; 1.1120x over previous
import functools

import jax
import jax.numpy as jnp
from jax import lax
from jax.experimental import pallas as pl

N_U = 100000
N_I = 100000
D = 64
Q = 5
L = 2
TEMP = 0.2
LAMBDA_1 = 0.2
LAMBDA_2 = 1e-07
B = 1024

_TILE = 2000


def _flash_body(a_ref, b_ref, c_ref, g_ref, o_ref):
    i = pl.program_id(0)

    @pl.when(i == 0)
    def _():
        o_ref[...] = jnp.zeros_like(o_ref)

    e = a_ref[...] + b_ref[...] + c_ref[...]
    nsq = jnp.sum(e * e, axis=1)
    scale = lax.rsqrt(jnp.maximum(nsq, 1e-24)) * (1.0 / TEMP)
    logits = lax.dot_general(g_ref[...], e, (((1,), (1,)), ((), ())),
                             preferred_element_type=jnp.float32)
    s = jnp.exp(logits * scale[None, :])
    o_ref[...] += jnp.sum(s, axis=1, keepdims=True)


def _flash_sum(tab_a, tab_b, tab_c, g_rows):
    n = tab_a.shape[0]
    grid = (n // _TILE,)
    out = pl.pallas_call(
        _flash_body,
        grid=grid,
        in_specs=[
            pl.BlockSpec((_TILE, D), lambda i: (i, 0)),
            pl.BlockSpec((_TILE, D), lambda i: (i, 0)),
            pl.BlockSpec((_TILE, D), lambda i: (i, 0)),
            pl.BlockSpec((B, D), lambda i: (0, 0)),
        ],
        out_specs=pl.BlockSpec((B, 128), lambda i: (0, 0)),
        out_shape=jax.ShapeDtypeStruct((B, 128), jnp.float32),
    )(tab_a, tab_b, tab_c, g_rows)
    return out[:, 0]


def _l2n(x):
    return x / jnp.maximum(jnp.linalg.norm(x, axis=-1, keepdims=True), 1e-12)


def kernel(uids, iids, pos, neg, adj_rows, adj_cols, adj_vals,
           E_u_0, E_i_0, u_mul_s, v_mul_s, ut, vt):
    f32 = jnp.float32
    Z_u1 = jax.ops.segment_sum(adj_vals[:, None] * E_i_0[adj_cols], adj_rows,
                               num_segments=N_U)
    Z_i1 = jax.ops.segment_sum(adj_vals[:, None] * E_u_0[adj_rows], adj_cols,
                               num_segments=N_I)
    Z_u2 = jax.ops.segment_sum(adj_vals[:, None] * Z_i1[adj_cols], adj_rows,
                               num_segments=N_U)
    Z_i2 = jax.ops.segment_sum(adj_vals[:, None] * Z_u1[adj_rows], adj_cols,
                               num_segments=N_I)

    S_u = vt @ (E_i_0 + Z_i1)
    S_i = ut @ (E_u_0 + Z_u1)

    eu0_u, zu1_u, zu2_u = E_u_0[uids], Z_u1[uids], Z_u2[uids]
    ei0_i, zi1_i, zi2_i = E_i_0[iids], Z_i1[iids], Z_i2[iids]
    ei0_p, zi1_p, zi2_p = E_i_0[pos], Z_i1[pos], Z_i2[pos]
    ei0_n, zi1_n, zi2_n = E_i_0[neg], Z_i1[neg], Z_i2[neg]

    gu_rows = _l2n(eu0_u + u_mul_s[uids] @ S_u)
    gi_rows = _l2n(ei0_i + v_mul_s[iids] @ S_i)

    sum_u = _flash_sum(E_u_0, Z_u1, Z_u2, gu_rows)
    sum_i = _flash_sum(E_i_0, Z_i1, Z_i2, gi_rows)
    neg_score = jnp.log(sum_u + 1e-08).mean() + jnp.log(sum_i + 1e-08).mean()

    eu_rows = eu0_u + zu1_u + zu2_u
    ei_rows = ei0_i + zi1_i + zi2_i
    pos_score = (jnp.clip((gu_rows * _l2n(eu_rows)).sum(1) / TEMP, -5.0, 5.0).mean()
                 + jnp.clip((gi_rows * _l2n(ei_rows)).sum(1) / TEMP, -5.0, 5.0).mean())
    loss_s = -pos_score + neg_score

    pos_emb = ei0_p + zi1_p + zi2_p
    neg_emb = ei0_n + zi1_n + zi2_n
    pos_scores = (eu_rows * pos_emb).sum(-1)
    neg_scores = (eu_rows * neg_emb).sum(-1)
    loss_r = -jnp.log(jax.nn.sigmoid(pos_scores - neg_scores)).mean()

    loss_reg = (jnp.sum(E_u_0.astype(f32) ** 2)
                + jnp.sum(E_i_0.astype(f32) ** 2)) * LAMBDA_2
    loss = loss_r + loss_reg + LAMBDA_1 * loss_s
    return (loss, loss_r, LAMBDA_1 * loss_s)
